# Initial kernel scaffold; baseline (speedup 1.0000x reference)
import jax
import jax.numpy as jnp
from jax.experimental import pallas as pl


def kernel(xyzs, original_xyzs, features, original_features, W_t, W_s1, W_s2):
    raise NotImplementedError("write your pallas kernel here")


# speedup vs baseline: 23.6392x; 23.6392x over previous
import jax
import jax.numpy as jnp
from jax import lax
from jax.experimental import pallas as pl

_EPS_BN = 1e-5
_EPS_D = 1e-8


def _knn_interp_body(xyzT_ref, anchT_ref, feat_ref, origf_ref, wt_ref,
                     newf_ref, acc_ref):
    b = pl.program_id(1)
    j = pl.program_id(2)
    nb = xyzT_ref[0, 0]
    an = anchT_ref[0, 0]
    F = jnp.maximum(
        jnp.dot(wt_ref[...], feat_ref[0, 0], preferred_element_type=jnp.float32),
        0.0)
    T = an.shape[1]
    N2 = nb.shape[1]
    cross = lax.dot_general(an, nb, (((0,), (0,)), ((), ())),
                            preferred_element_type=jnp.float32)
    a2 = jnp.sum(an * an, axis=0)
    b2 = jnp.sum(nb * nb, axis=0)
    d2 = a2[:, None] + b2[None, :] - 2.0 * cross

    iota = lax.broadcasted_iota(jnp.int32, (T, N2), 1)
    cur = d2
    ds = []
    idxs = []
    for _ in range(3):
        m = jnp.min(cur, axis=1)
        idx = jnp.min(jnp.where(cur == m[:, None], iota, N2), axis=1)
        ds.append(m)
        idxs.append(idx)
        cur = jnp.where(iota == idx[:, None], jnp.float32(1e30), cur)

    rs = [1.0 / (jnp.sqrt(jnp.maximum(d, 0.0)) + _EPS_D) for d in ds]
    norm = rs[0] + rs[1] + rs[2]
    S = jnp.zeros((T, N2), jnp.float32)
    for k in range(3):
        w = rs[k] / norm
        S = S + jnp.where(iota == idxs[k][:, None], w[:, None], 0.0)
    interp = lax.dot_general(F, S, (((1,), (1,)), ((), ())),
                             preferred_element_type=jnp.float32)
    newf = jnp.concatenate([interp, origf_ref[0, 0]], axis=0)
    newf_ref[0, 0] = newf

    faug = jnp.concatenate([newf, jnp.ones((8, T), jnp.float32)], axis=0)
    contrib = lax.dot_general(faug, faug, (((1,), (1,)), ((), ())),
                              preferred_element_type=jnp.float32)
    first = jnp.logical_and(b == 0, j == 0)

    @pl.when(first)
    def _():
        acc_ref[0] = contrib

    @pl.when(jnp.logical_not(first))
    def _():
        acc_ref[0] = acc_ref[0] + contrib


def _stats2_body(newf_ref, a1_ref, acc_ref):
    b = pl.program_id(1)
    j = pl.program_id(2)
    newf = newf_ref[0, 0]
    T = newf.shape[1]
    faug = jnp.concatenate([newf, jnp.ones((8, T), jnp.float32)], axis=0)
    x = jnp.maximum(
        lax.dot_general(a1_ref[0], faug, (((1,), (0,)), ((), ())),
                        preferred_element_type=jnp.float32), 0.0)
    xaug = jnp.concatenate([x, jnp.ones((8, T), jnp.float32)], axis=0)
    contrib = lax.dot_general(xaug, xaug, (((1,), (1,)), ((), ())),
                              preferred_element_type=jnp.float32)
    first = jnp.logical_and(b == 0, j == 0)

    @pl.when(first)
    def _():
        acc_ref[0] = contrib

    @pl.when(jnp.logical_not(first))
    def _():
        acc_ref[0] = acc_ref[0] + contrib


def _final_body(newf_ref, a1_ref, a2_ref, out_ref):
    newf = newf_ref[0, 0]
    T = newf.shape[1]
    faug = jnp.concatenate([newf, jnp.ones((8, T), jnp.float32)], axis=0)
    x = jnp.maximum(
        lax.dot_general(a1_ref[0], faug, (((1,), (0,)), ((), ())),
                        preferred_element_type=jnp.float32), 0.0)
    xaug = jnp.concatenate([x, jnp.ones((8, T), jnp.float32)], axis=0)
    y = jnp.maximum(
        lax.dot_general(a2_ref[0], xaug, (((1,), (0,)), ((), ())),
                        preferred_element_type=jnp.float32), 0.0)
    out_ref[0, 0] = y


def _fold_bn(W, M, s, n):
    mean = jnp.einsum('ok,lk->lo', W, s) / n
    WM = jnp.einsum('ok,lkm->lom', W, M)
    ey2 = jnp.einsum('lom,om->lo', WM, W) / n
    var = ey2 - mean * mean
    sc = lax.rsqrt(var + _EPS_BN)
    A = W[None, :, :] * sc[:, :, None]
    bias = -mean * sc
    baug = jnp.repeat((bias / 8.0)[:, :, None], 8, axis=2)
    return jnp.concatenate([A, baug], axis=2)


def kernel(xyzs, original_xyzs, features, original_features, W_t, W_s1, W_s2):
    B, L, N2, _ = xyzs.shape
    N1 = original_xyzs.shape[2]
    Cin = features.shape[2]
    Corig = original_features.shape[2]
    T = 256 if N1 % 256 == 0 else N1
    J = N1 // T
    f32 = jnp.float32

    xyzT = jnp.swapaxes(xyzs, 2, 3)
    anchT = jnp.swapaxes(original_xyzs, 2, 3)

    grid = (L, B, J)
    newf, acc1 = pl.pallas_call(
        _knn_interp_body,
        grid=grid,
        in_specs=[
            pl.BlockSpec((1, 1, 3, N2), lambda t, b, j: (b, t, 0, 0)),
            pl.BlockSpec((1, 1, 3, T), lambda t, b, j: (b, t, 0, j)),
            pl.BlockSpec((1, 1, Cin, N2), lambda t, b, j: (b, t, 0, 0)),
            pl.BlockSpec((1, 1, Corig, T), lambda t, b, j: (b, t, 0, j)),
            pl.BlockSpec((Cin, Cin), lambda t, b, j: (0, 0)),
        ],
        out_specs=[
            pl.BlockSpec((1, 1, Cin + Corig, T), lambda t, b, j: (b, t, 0, j)),
            pl.BlockSpec((1, 104, 104), lambda t, b, j: (t, 0, 0)),
        ],
        out_shape=[
            jax.ShapeDtypeStruct((B, L, Cin + Corig, N1), f32),
            jax.ShapeDtypeStruct((L, 104, 104), f32),
        ],
    )(xyzT, anchT, features, original_features, W_t)

    n = B * N1
    M1 = acc1[:, :96, :96]
    s1 = acc1[:, 96, :96]
    A1aug = _fold_bn(W_s1, M1, s1, n)

    acc2 = pl.pallas_call(
        _stats2_body,
        grid=grid,
        in_specs=[
            pl.BlockSpec((1, 1, 96, T), lambda t, b, j: (b, t, 0, j)),
            pl.BlockSpec((1, 128, 104), lambda t, b, j: (t, 0, 0)),
        ],
        out_specs=pl.BlockSpec((1, 136, 136), lambda t, b, j: (t, 0, 0)),
        out_shape=jax.ShapeDtypeStruct((L, 136, 136), f32),
    )(newf, A1aug)

    M2 = acc2[:, :128, :128]
    s2 = acc2[:, 128, :128]
    A2aug = _fold_bn(W_s2, M2, s2, n)

    out = pl.pallas_call(
        _final_body,
        grid=grid,
        in_specs=[
            pl.BlockSpec((1, 1, 96, T), lambda t, b, j: (b, t, 0, j)),
            pl.BlockSpec((1, 128, 104), lambda t, b, j: (t, 0, 0)),
            pl.BlockSpec((1, 128, 136), lambda t, b, j: (t, 0, 0)),
        ],
        out_specs=pl.BlockSpec((1, 1, 128, T), lambda t, b, j: (b, t, 0, j)),
        out_shape=jax.ShapeDtypeStruct((B, L, 128, N1), f32),
    )(newf, A1aug, A2aug)
    return out
